# baseline (speedup 1.0000x reference)
import jax
import jax.numpy as jnp
from jax import lax
from jax.experimental import pallas as pl
from jax.experimental.pallas import tpu as pltpu
from jax.experimental.pallas import tpu_sc as plsc

B = 16384
D = 64
NC = 2
NS = 16
NW = NC * NS
BPW = B // NW
CHUNK = 128
NCH = BPW // CHUNK
GPC = CHUNK // 16

N_U = 1000000
N_I = 100000
CB_U = (N_U + 127) // 128
CB_I = (N_I + 127) // 128


def _transpose_stripe(colbuf, rowbuf):
    iota16 = lax.iota(jnp.int32, 16)
    lanes_l0 = [l0 + iota16 for l0 in range(0, 128, 16)]
    v_l0 = [(l0 + iota16) & 63 for l0 in range(0, 128, 16)]
    hb_l0 = [(l0 // 64) * D for l0 in range(0, 128, 16)]

    def r_body(r, _):
        for j in range(2):
            rowoff = (iota16 + (2 * r + j)) & 15
            for d0 in range(0, D, 16):
                rows = rowoff + d0
                vals = [plsc.load_gather(colbuf, [rows, lanes_l0[q]])
                        for q in range(8)]
                for q in range(8):
                    plsc.store_scatter(rowbuf, [v_l0[q], rows + hb_l0[q]],
                                       vals[q])
        return 0

    lax.fori_loop(0, 8, r_body, 0)


def _repack_body(it_t, i2_out,
                 colA, colB, rowA, rowB, semA, semB):
    c = lax.axis_index("c")
    s = lax.axis_index("s")
    wid = s * NC + c

    def make_pair_loop(src, dst, n_full):
        def pair_body(tp, _):
            cb0 = wid + (2 * tp) * NW
            cb1 = wid + (2 * tp + 1) * NW
            cb0c = jnp.minimum(cb0, n_full - 1)
            cb1c = jnp.minimum(cb1, n_full - 1)
            st0 = pl.multiple_of(cb0c * 128, 128)
            st1 = pl.multiple_of(cb1c * 128, 128)
            ci0 = pltpu.async_copy(src.at[:, pl.ds(st0, 128)], colA, semA)
            ci1 = pltpu.async_copy(src.at[:, pl.ds(st1, 128)], colB, semB)
            ci0.wait()
            _transpose_stripe(colA, rowA)
            co0 = pltpu.async_copy(rowA, dst.at[pl.ds(cb0c * 64, 64)], semA)
            ci1.wait()
            _transpose_stripe(colB, rowB)
            co1 = pltpu.async_copy(rowB, dst.at[pl.ds(cb1c * 64, 64)], semB)
            co0.wait()
            co1.wait()
            return 0

        return pair_body

    n_full_i = N_I // 128
    lax.fori_loop(0, (n_full_i + 2 * NW - 1) // (2 * NW),
                  make_pair_loop(it_t, i2_out, n_full_i), 0)

    def tail(src, dst, cb, n_rows, who):
        @pl.when(wid == who)
        def _():
            start = pl.multiple_of(cb * 128, 128)
            pltpu.sync_copy(src.at[:, pl.ds(start, 128)], colA)
            _transpose_stripe(colA, rowA)
            nv = n_rows - cb * 128
            for q in range(4):
                if True:
                    @pl.when(nv >= (q + 1) * 16)
                    def _(q=q):
                        pltpu.sync_copy(rowA.at[pl.ds(q * 16, 16)],
                                        dst.at[pl.ds(cb * 64 + q * 16, 16)])

    tail(it_t, i2_out, n_full_i, N_I, 1)


@jax.jit
def _repack(it_t):
    mesh = plsc.VectorSubcoreMesh(core_axis_name="c", subcore_axis_name="s",
                                  num_cores=NC, num_subcores=NS)
    f32 = jnp.float32
    run = pl.kernel(
        _repack_body,
        out_type=(jax.ShapeDtypeStruct((CB_I * 64, 128), f32),),
        mesh=mesh,
        scratch_types=[
            pltpu.VMEM((D, 128), f32),
            pltpu.VMEM((D, 128), f32),
            pltpu.VMEM((D, 128), f32),
            pltpu.VMEM((D, 128), f32),
            pltpu.SemaphoreType.DMA,
            pltpu.SemaphoreType.DMA,
        ],
        compiler_params=pltpu.CompilerParams(
            needs_layout_passes=False,
            use_tc_tiling_on_sc=True,
            disable_bounds_checks=True),
    )
    return run(it_t)


def _tc_transpose_body(x_ref, o_ref):
    xt = jnp.swapaxes(x_ref[...], 0, 1)
    o_ref[...] = jnp.concatenate([xt[0:D, :], xt[D:2 * D, :]], axis=1)


@jax.jit
def _tc_transpose(ut_t):
    return pl.pallas_call(
        _tc_transpose_body,
        grid=(CB_U,),
        in_specs=[pl.BlockSpec((D, 128), lambda j: (0, j))],
        out_specs=pl.BlockSpec((D, 2 * D), lambda j: (j, 0)),
        out_shape=jax.ShapeDtypeStruct((CB_U * 64, 2 * D), jnp.float32),
    )(ut_t)


def _bprmf_body(users_r, pos_r, neg_r, ut_r, it_r,
                pos_s_out, neg_s_out, u_out, p_out, n_out,
                idx_u, idx_p, idx_n, vr_u, vr_p, vr_n,
                u_rows, p_rows, n_rows, sc_p, sc_n, sem):
    c = lax.axis_index("c")
    s = lax.axis_index("s")
    wid = s * NC + c
    base = wid * BPW

    pltpu.sync_copy(users_r.at[pl.ds(wid * NCH, NCH)], idx_u)
    pltpu.sync_copy(pos_r.at[pl.ds(wid * NCH, NCH)], idx_p)
    pltpu.sync_copy(neg_r.at[pl.ds(wid * NCH, NCH)], idx_n)

    for ch in range(NCH):
        for g in range(GPC):
            sl = pl.ds(g * 16, 16)
            vr_u[ch, sl] = ((idx_u[ch, sl] >> 7) << 6) + (idx_u[ch, sl] & 63)
            vr_p[ch, sl] = ((idx_p[ch, sl] >> 7) << 6) + (idx_p[ch, sl] & 63)
            vr_n[ch, sl] = ((idx_n[ch, sl] >> 7) << 6) + (idx_n[ch, sl] & 63)

    iota16 = lax.iota(jnp.int32, 16)
    zero16 = jnp.zeros((16,), jnp.float32)

    for ch in range(NCH):
        cps = (pltpu.async_copy(ut_r.at[vr_u.at[ch]], u_rows, sem),
               pltpu.async_copy(it_r.at[vr_p.at[ch]], p_rows, sem),
               pltpu.async_copy(it_r.at[vr_n.at[ch]], n_rows, sem))
        for cp in cps:
            cp.wait()

        for g in range(GPC):
            sl = pl.ds(g * 16, 16)
            rows_e = g * 16 + iota16
            hu = ((idx_u[ch, sl] >> 6) & 1) * D
            hp = ((idx_p[ch, sl] >> 6) & 1) * D
            hn = ((idx_n[ch, sl] >> 6) & 1) * D

            def dbody(d, carry, rows_e=rows_e, hu=hu, hp=hp, hn=hn):
                ap, an = carry
                uc = plsc.load_gather(u_rows, [rows_e, hu + d])
                pc = plsc.load_gather(p_rows, [rows_e, hp + d])
                nc = plsc.load_gather(n_rows, [rows_e, hn + d])
                return (ap + uc * pc, an + uc * nc)

            ap, an = lax.fori_loop(0, D, dbody, (zero16, zero16))
            osl = pl.ds(ch * CHUNK + g * 16, 16)
            sc_p[osl] = ap
            sc_n[osl] = an

        out_sl = pl.ds(base + ch * CHUNK, CHUNK)
        pltpu.sync_copy(u_rows, u_out.at[out_sl])
        pltpu.sync_copy(p_rows, p_out.at[out_sl])
        pltpu.sync_copy(n_rows, n_out.at[out_sl])

    out_sl = pl.ds(base, BPW)
    pltpu.sync_copy(sc_p, pos_s_out.at[out_sl])
    pltpu.sync_copy(sc_n, neg_s_out.at[out_sl])


@jax.jit
def _bprmf(users2, pos2, neg2, ut2, it2):
    mesh = plsc.VectorSubcoreMesh(core_axis_name="c", subcore_axis_name="s",
                                  num_cores=NC, num_subcores=NS)
    f32 = jnp.float32
    out_type = (
        jax.ShapeDtypeStruct((B,), f32),
        jax.ShapeDtypeStruct((B,), f32),
        jax.ShapeDtypeStruct((B, 2 * D), f32),
        jax.ShapeDtypeStruct((B, 2 * D), f32),
        jax.ShapeDtypeStruct((B, 2 * D), f32),
    )
    i32 = jnp.int32
    scratch = [
        pltpu.VMEM((NCH, CHUNK), i32),
        pltpu.VMEM((NCH, CHUNK), i32),
        pltpu.VMEM((NCH, CHUNK), i32),
        pltpu.VMEM((NCH, CHUNK), i32),
        pltpu.VMEM((NCH, CHUNK), i32),
        pltpu.VMEM((NCH, CHUNK), i32),
        pltpu.VMEM((CHUNK, 2 * D), f32),
        pltpu.VMEM((CHUNK, 2 * D), f32),
        pltpu.VMEM((CHUNK, 2 * D), f32),
        pltpu.VMEM((BPW,), f32),
        pltpu.VMEM((BPW,), f32),
        pltpu.SemaphoreType.DMA,
    ]
    run = pl.kernel(_bprmf_body, out_type=out_type, mesh=mesh,
                    scratch_types=scratch,
                    compiler_params=pltpu.CompilerParams(
                        needs_layout_passes=False,
                        use_tc_tiling_on_sc=True))
    return run(users2, pos2, neg2, ut2, it2)


def kernel(users, pos_items, neg_items, user_table, item_table):
    users2 = users.astype(jnp.int32).reshape(NW * NCH, CHUNK)
    pos2 = pos_items.astype(jnp.int32).reshape(NW * NCH, CHUNK)
    neg2 = neg_items.astype(jnp.int32).reshape(NW * NCH, CHUNK)
    ut2 = _tc_transpose(user_table.T)
    (it2,) = _repack(item_table.T)
    ps, ns, uv, pv, nv = _bprmf(users2, pos2, neg2, ut2, it2)
    u_odd = ((users.astype(jnp.int32) >> 6) & 1)[:, None] == 1
    p_odd = ((pos_items.astype(jnp.int32) >> 6) & 1)[:, None] == 1
    n_odd = ((neg_items.astype(jnp.int32) >> 6) & 1)[:, None] == 1
    u_emb = jnp.where(u_odd, uv[:, D:], uv[:, :D])
    pos_emb = jnp.where(p_odd, pv[:, D:], pv[:, :D])
    neg_emb = jnp.where(n_odd, nv[:, D:], nv[:, :D])
    return (ps, ns, u_emb, pos_emb, neg_emb)


# speedup vs baseline: 5.1963x; 5.1963x over previous
import jax
import jax.numpy as jnp
from jax import lax
from jax.experimental import pallas as pl
from jax.experimental.pallas import tpu as pltpu
from jax.experimental.pallas import tpu_sc as plsc

B = 16384
D = 64
NC = 2
NS = 16
NW = NC * NS
BPW = B // NW
CHUNK = 128
NCH = BPW // CHUNK
GPC = CHUNK // 16

N_U = 1000000
N_I = 100000
CB_U = (N_U + 127) // 128
CB_I = (N_I + 127) // 128


def _transpose_stripe(colbuf, rowbuf):
    iota16 = lax.iota(jnp.int32, 16)
    lanes_l0 = [l0 + iota16 for l0 in range(0, 128, 16)]
    v_l0 = [(l0 + iota16) & 63 for l0 in range(0, 128, 16)]
    hb_l0 = [(l0 // 64) * D for l0 in range(0, 128, 16)]

    def r_body(r, _):
        for j in range(2):
            rowoff = (iota16 + (2 * r + j)) & 15
            for d0 in range(0, D, 16):
                rows = rowoff + d0
                vals = [plsc.load_gather(colbuf, [rows, lanes_l0[q]])
                        for q in range(8)]
                for q in range(8):
                    plsc.store_scatter(rowbuf, [v_l0[q], rows + hb_l0[q]],
                                       vals[q])
        return 0

    lax.fori_loop(0, 8, r_body, 0)


def _repack_body(it_t, i2_out,
                 colA, colB, rowA, rowB, semA, semB):
    c = lax.axis_index("c")
    s = lax.axis_index("s")
    wid = s * NC + c

    def make_pair_loop(src, dst, n_full):
        def pair_body(tp, _):
            cb0 = wid + (2 * tp) * NW
            cb1 = wid + (2 * tp + 1) * NW
            cb0c = jnp.minimum(cb0, n_full - 1)
            cb1c = jnp.minimum(cb1, n_full - 1)
            st0 = pl.multiple_of(cb0c * 128, 128)
            st1 = pl.multiple_of(cb1c * 128, 128)
            ci0 = pltpu.async_copy(src.at[:, pl.ds(st0, 128)], colA, semA)
            ci1 = pltpu.async_copy(src.at[:, pl.ds(st1, 128)], colB, semB)
            ci0.wait()
            _transpose_stripe(colA, rowA)
            co0 = pltpu.async_copy(rowA, dst.at[pl.ds(cb0c * 64, 64)], semA)
            ci1.wait()
            _transpose_stripe(colB, rowB)
            co1 = pltpu.async_copy(rowB, dst.at[pl.ds(cb1c * 64, 64)], semB)
            co0.wait()
            co1.wait()
            return 0

        return pair_body

    n_full_i = N_I // 128
    lax.fori_loop(0, (n_full_i + 2 * NW - 1) // (2 * NW),
                  make_pair_loop(it_t, i2_out, n_full_i), 0)

    def tail(src, dst, cb, n_rows, who):
        @pl.when(wid == who)
        def _():
            start = pl.multiple_of(cb * 128, 128)
            pltpu.sync_copy(src.at[:, pl.ds(start, 128)], colA)
            _transpose_stripe(colA, rowA)
            nv = n_rows - cb * 128
            for q in range(4):
                if True:
                    @pl.when(nv >= (q + 1) * 16)
                    def _(q=q):
                        pltpu.sync_copy(rowA.at[pl.ds(q * 16, 16)],
                                        dst.at[pl.ds(cb * 64 + q * 16, 16)])

    tail(it_t, i2_out, n_full_i, N_I, 1)


@jax.jit
def _repack(it_t):
    mesh = plsc.VectorSubcoreMesh(core_axis_name="c", subcore_axis_name="s",
                                  num_cores=NC, num_subcores=NS)
    f32 = jnp.float32
    run = pl.kernel(
        _repack_body,
        out_type=(jax.ShapeDtypeStruct((CB_I * 64, 128), f32),),
        mesh=mesh,
        scratch_types=[
            pltpu.VMEM((D, 128), f32),
            pltpu.VMEM((D, 128), f32),
            pltpu.VMEM((D, 128), f32),
            pltpu.VMEM((D, 128), f32),
            pltpu.SemaphoreType.DMA,
            pltpu.SemaphoreType.DMA,
        ],
        compiler_params=pltpu.CompilerParams(
            needs_layout_passes=False,
            use_tc_tiling_on_sc=True,
            disable_bounds_checks=True),
    )
    return run(it_t)


SPB = 8


def _tc_transpose_body(x_ref, o_ref):
    xt = jnp.swapaxes(x_ref[...], 0, 1)
    lo = jnp.concatenate([xt[sl * 128:sl * 128 + D, :] for sl in range(SPB)],
                         axis=0)
    hi = jnp.concatenate([xt[sl * 128 + D:(sl + 1) * 128, :]
                          for sl in range(SPB)], axis=0)
    o_ref[...] = jnp.concatenate([lo, hi], axis=1)


@jax.jit
def _tc_transpose(ut_t):
    nblk = (CB_U + SPB - 1) // SPB
    return pl.pallas_call(
        _tc_transpose_body,
        grid=(nblk,),
        in_specs=[pl.BlockSpec((D, SPB * 128), lambda j: (0, j))],
        out_specs=pl.BlockSpec((SPB * D, 2 * D), lambda j: (j, 0)),
        out_shape=jax.ShapeDtypeStruct((CB_U * 64, 2 * D), jnp.float32),
    )(ut_t)


def _bprmf_body(users_r, pos_r, neg_r, ut_r, it_r,
                pos_s_out, neg_s_out, u_out, p_out, n_out,
                idx_u, idx_p, idx_n, vr_u, vr_p, vr_n,
                u_rows, p_rows, n_rows, sc_p, sc_n, sem):
    c = lax.axis_index("c")
    s = lax.axis_index("s")
    wid = s * NC + c
    base = wid * BPW

    pltpu.sync_copy(users_r.at[pl.ds(wid * NCH, NCH)], idx_u)
    pltpu.sync_copy(pos_r.at[pl.ds(wid * NCH, NCH)], idx_p)
    pltpu.sync_copy(neg_r.at[pl.ds(wid * NCH, NCH)], idx_n)

    for ch in range(NCH):
        for g in range(GPC):
            sl = pl.ds(g * 16, 16)
            vr_u[ch, sl] = ((idx_u[ch, sl] >> 7) << 6) + (idx_u[ch, sl] & 63)
            vr_p[ch, sl] = ((idx_p[ch, sl] >> 7) << 6) + (idx_p[ch, sl] & 63)
            vr_n[ch, sl] = ((idx_n[ch, sl] >> 7) << 6) + (idx_n[ch, sl] & 63)

    iota16 = lax.iota(jnp.int32, 16)
    zero16 = jnp.zeros((16,), jnp.float32)

    for ch in range(NCH):
        cps = (pltpu.async_copy(ut_r.at[vr_u.at[ch]], u_rows, sem),
               pltpu.async_copy(it_r.at[vr_p.at[ch]], p_rows, sem),
               pltpu.async_copy(it_r.at[vr_n.at[ch]], n_rows, sem))
        for cp in cps:
            cp.wait()

        for g in range(GPC):
            sl = pl.ds(g * 16, 16)
            rows_e = g * 16 + iota16
            hu = ((idx_u[ch, sl] >> 6) & 1) * D
            hp = ((idx_p[ch, sl] >> 6) & 1) * D
            hn = ((idx_n[ch, sl] >> 6) & 1) * D

            def dbody(d, carry, rows_e=rows_e, hu=hu, hp=hp, hn=hn):
                ap, an = carry
                uc = plsc.load_gather(u_rows, [rows_e, hu + d])
                pc = plsc.load_gather(p_rows, [rows_e, hp + d])
                nc = plsc.load_gather(n_rows, [rows_e, hn + d])
                return (ap + uc * pc, an + uc * nc)

            ap, an = lax.fori_loop(0, D, dbody, (zero16, zero16))
            osl = pl.ds(ch * CHUNK + g * 16, 16)
            sc_p[osl] = ap
            sc_n[osl] = an

        out_sl = pl.ds(base + ch * CHUNK, CHUNK)
        pltpu.sync_copy(u_rows, u_out.at[out_sl])
        pltpu.sync_copy(p_rows, p_out.at[out_sl])
        pltpu.sync_copy(n_rows, n_out.at[out_sl])

    out_sl = pl.ds(base, BPW)
    pltpu.sync_copy(sc_p, pos_s_out.at[out_sl])
    pltpu.sync_copy(sc_n, neg_s_out.at[out_sl])


@jax.jit
def _bprmf(users2, pos2, neg2, ut2, it2):
    mesh = plsc.VectorSubcoreMesh(core_axis_name="c", subcore_axis_name="s",
                                  num_cores=NC, num_subcores=NS)
    f32 = jnp.float32
    out_type = (
        jax.ShapeDtypeStruct((B,), f32),
        jax.ShapeDtypeStruct((B,), f32),
        jax.ShapeDtypeStruct((B, 2 * D), f32),
        jax.ShapeDtypeStruct((B, 2 * D), f32),
        jax.ShapeDtypeStruct((B, 2 * D), f32),
    )
    i32 = jnp.int32
    scratch = [
        pltpu.VMEM((NCH, CHUNK), i32),
        pltpu.VMEM((NCH, CHUNK), i32),
        pltpu.VMEM((NCH, CHUNK), i32),
        pltpu.VMEM((NCH, CHUNK), i32),
        pltpu.VMEM((NCH, CHUNK), i32),
        pltpu.VMEM((NCH, CHUNK), i32),
        pltpu.VMEM((CHUNK, 2 * D), f32),
        pltpu.VMEM((CHUNK, 2 * D), f32),
        pltpu.VMEM((CHUNK, 2 * D), f32),
        pltpu.VMEM((BPW,), f32),
        pltpu.VMEM((BPW,), f32),
        pltpu.SemaphoreType.DMA,
    ]
    run = pl.kernel(_bprmf_body, out_type=out_type, mesh=mesh,
                    scratch_types=scratch,
                    compiler_params=pltpu.CompilerParams(
                        needs_layout_passes=False,
                        use_tc_tiling_on_sc=True))
    return run(users2, pos2, neg2, ut2, it2)


def kernel(users, pos_items, neg_items, user_table, item_table):
    users2 = users.astype(jnp.int32).reshape(NW * NCH, CHUNK)
    pos2 = pos_items.astype(jnp.int32).reshape(NW * NCH, CHUNK)
    neg2 = neg_items.astype(jnp.int32).reshape(NW * NCH, CHUNK)
    ut2 = _tc_transpose(user_table.T)
    (it2,) = _repack(item_table.T)
    ps, ns, uv, pv, nv = _bprmf(users2, pos2, neg2, ut2, it2)
    u_odd = ((users.astype(jnp.int32) >> 6) & 1)[:, None] == 1
    p_odd = ((pos_items.astype(jnp.int32) >> 6) & 1)[:, None] == 1
    n_odd = ((neg_items.astype(jnp.int32) >> 6) & 1)[:, None] == 1
    u_emb = jnp.where(u_odd, uv[:, D:], uv[:, :D])
    pos_emb = jnp.where(p_odd, pv[:, D:], pv[:, :D])
    neg_emb = jnp.where(n_odd, nv[:, D:], nv[:, :D])
    return (ps, ns, u_emb, pos_emb, neg_emb)


# speedup vs baseline: 7.4720x; 1.4380x over previous
import jax
import jax.numpy as jnp
from jax import lax
from jax.experimental import pallas as pl
from jax.experimental.pallas import tpu as pltpu
from jax.experimental.pallas import tpu_sc as plsc

B = 16384
D = 64
NC = 2
NS = 16
NW = NC * NS
BPW = B // NW
CHUNK = 128
NCH = BPW // CHUNK
GPC = CHUNK // 16

N_U = 1000000
N_I = 100000
CB_U = (N_U + 127) // 128
CB_I = (N_I + 127) // 128


def _transpose_stripe(colbuf, rowbuf):
    iota16 = lax.iota(jnp.int32, 16)
    lanes_l0 = [l0 + iota16 for l0 in range(0, 128, 16)]
    v_l0 = [(l0 + iota16) & 63 for l0 in range(0, 128, 16)]
    hb_l0 = [(l0 // 64) * D for l0 in range(0, 128, 16)]

    def r_body(r, _):
        for j in range(2):
            rowoff = (iota16 + (2 * r + j)) & 15
            for d0 in range(0, D, 16):
                rows = rowoff + d0
                vals = [plsc.load_gather(colbuf, [rows, lanes_l0[q]])
                        for q in range(8)]
                for q in range(8):
                    plsc.store_scatter(rowbuf, [v_l0[q], rows + hb_l0[q]],
                                       vals[q])
        return 0

    lax.fori_loop(0, 8, r_body, 0)


def _repack_body(ut_t, it_t, u2_out, i2_out,
                 colA, colB, rowA, rowB, semA, semB):
    c = lax.axis_index("c")
    s = lax.axis_index("s")
    wid = s * NC + c

    def make_pair_loop(src, dst, n_full):
        def pair_body(tp, _):
            cb0 = wid + (2 * tp) * NW
            cb1 = wid + (2 * tp + 1) * NW
            cb0c = jnp.minimum(cb0, n_full - 1)
            cb1c = jnp.minimum(cb1, n_full - 1)
            st0 = pl.multiple_of(cb0c * 128, 128)
            st1 = pl.multiple_of(cb1c * 128, 128)
            ci0 = pltpu.async_copy(src.at[:, pl.ds(st0, 128)], colA, semA)
            ci1 = pltpu.async_copy(src.at[:, pl.ds(st1, 128)], colB, semB)
            ci0.wait()
            _transpose_stripe(colA, rowA)
            co0 = pltpu.async_copy(rowA, dst.at[pl.ds(cb0c * 64, 64)], semA)
            ci1.wait()
            _transpose_stripe(colB, rowB)
            co1 = pltpu.async_copy(rowB, dst.at[pl.ds(cb1c * 64, 64)], semB)
            co0.wait()
            co1.wait()
            return 0

        return pair_body

    n_full_u = N_U // 128
    n_full_i = N_I // 128
    lax.fori_loop(0, (n_full_u + 2 * NW - 1) // (2 * NW),
                  make_pair_loop(ut_t, u2_out, n_full_u), 0)
    lax.fori_loop(0, (n_full_i + 2 * NW - 1) // (2 * NW),
                  make_pair_loop(it_t, i2_out, n_full_i), 0)

    def tail(src, dst, cb, n_rows, who):
        @pl.when(wid == who)
        def _():
            start = pl.multiple_of(cb * 128, 128)
            pltpu.sync_copy(src.at[:, pl.ds(start, 128)], colA)
            _transpose_stripe(colA, rowA)
            nv = n_rows - cb * 128
            for q in range(4):
                @pl.when(nv >= (q + 1) * 16)
                def _(q=q):
                    pltpu.sync_copy(rowA.at[pl.ds(q * 16, 16)],
                                    dst.at[pl.ds(cb * 64 + q * 16, 16)])

    tail(ut_t, u2_out, n_full_u, N_U, 0)
    tail(it_t, i2_out, n_full_i, N_I, 1)


@jax.jit
def _repack(ut_t, it_t):
    mesh = plsc.VectorSubcoreMesh(core_axis_name="c", subcore_axis_name="s",
                                  num_cores=NC, num_subcores=NS)
    f32 = jnp.float32
    run = pl.kernel(
        _repack_body,
        out_type=(jax.ShapeDtypeStruct((CB_U * 64, 128), f32),
                  jax.ShapeDtypeStruct((CB_I * 64, 128), f32)),
        mesh=mesh,
        scratch_types=[
            pltpu.VMEM((D, 128), f32),
            pltpu.VMEM((D, 128), f32),
            pltpu.VMEM((D, 128), f32),
            pltpu.VMEM((D, 128), f32),
            pltpu.SemaphoreType.DMA,
            pltpu.SemaphoreType.DMA,
        ],
        compiler_params=pltpu.CompilerParams(
            needs_layout_passes=False,
            use_tc_tiling_on_sc=True,
            disable_bounds_checks=True),
    )
    return run(ut_t, it_t)


def _bprmf_body(users_r, pos_r, neg_r, ut_r, it_r,
                pos_s_out, neg_s_out, u_out, p_out, n_out,
                idx_u, idx_p, idx_n, vr_u, vr_p, vr_n,
                u_rows, p_rows, n_rows, sc_p, sc_n, sem):
    c = lax.axis_index("c")
    s = lax.axis_index("s")
    wid = s * NC + c
    base = wid * BPW

    pltpu.sync_copy(users_r.at[pl.ds(wid * NCH, NCH)], idx_u)
    pltpu.sync_copy(pos_r.at[pl.ds(wid * NCH, NCH)], idx_p)
    pltpu.sync_copy(neg_r.at[pl.ds(wid * NCH, NCH)], idx_n)

    for ch in range(NCH):
        for g in range(GPC):
            sl = pl.ds(g * 16, 16)
            vr_u[ch, sl] = ((idx_u[ch, sl] >> 7) << 6) + (idx_u[ch, sl] & 63)
            vr_p[ch, sl] = ((idx_p[ch, sl] >> 7) << 6) + (idx_p[ch, sl] & 63)
            vr_n[ch, sl] = ((idx_n[ch, sl] >> 7) << 6) + (idx_n[ch, sl] & 63)

    iota16 = lax.iota(jnp.int32, 16)
    zero16 = jnp.zeros((16,), jnp.float32)

    for ch in range(NCH):
        cps = (pltpu.async_copy(ut_r.at[vr_u.at[ch]], u_rows, sem),
               pltpu.async_copy(it_r.at[vr_p.at[ch]], p_rows, sem),
               pltpu.async_copy(it_r.at[vr_n.at[ch]], n_rows, sem))
        for cp in cps:
            cp.wait()

        for g in range(GPC):
            sl = pl.ds(g * 16, 16)
            rows_e = g * 16 + iota16
            hu = ((idx_u[ch, sl] >> 6) & 1) * D
            hp = ((idx_p[ch, sl] >> 6) & 1) * D
            hn = ((idx_n[ch, sl] >> 6) & 1) * D

            def dbody(d, carry, rows_e=rows_e, hu=hu, hp=hp, hn=hn):
                ap, an = carry
                uc = plsc.load_gather(u_rows, [rows_e, hu + d])
                pc = plsc.load_gather(p_rows, [rows_e, hp + d])
                nc = plsc.load_gather(n_rows, [rows_e, hn + d])
                return (ap + uc * pc, an + uc * nc)

            ap, an = lax.fori_loop(0, D, dbody, (zero16, zero16))
            osl = pl.ds(ch * CHUNK + g * 16, 16)
            sc_p[osl] = ap
            sc_n[osl] = an

        out_sl = pl.ds(base + ch * CHUNK, CHUNK)
        pltpu.sync_copy(u_rows, u_out.at[out_sl])
        pltpu.sync_copy(p_rows, p_out.at[out_sl])
        pltpu.sync_copy(n_rows, n_out.at[out_sl])

    out_sl = pl.ds(base, BPW)
    pltpu.sync_copy(sc_p, pos_s_out.at[out_sl])
    pltpu.sync_copy(sc_n, neg_s_out.at[out_sl])


@jax.jit
def _bprmf(users2, pos2, neg2, ut2, it2):
    mesh = plsc.VectorSubcoreMesh(core_axis_name="c", subcore_axis_name="s",
                                  num_cores=NC, num_subcores=NS)
    f32 = jnp.float32
    out_type = (
        jax.ShapeDtypeStruct((B,), f32),
        jax.ShapeDtypeStruct((B,), f32),
        jax.ShapeDtypeStruct((B, 2 * D), f32),
        jax.ShapeDtypeStruct((B, 2 * D), f32),
        jax.ShapeDtypeStruct((B, 2 * D), f32),
    )
    i32 = jnp.int32
    scratch = [
        pltpu.VMEM((NCH, CHUNK), i32),
        pltpu.VMEM((NCH, CHUNK), i32),
        pltpu.VMEM((NCH, CHUNK), i32),
        pltpu.VMEM((NCH, CHUNK), i32),
        pltpu.VMEM((NCH, CHUNK), i32),
        pltpu.VMEM((NCH, CHUNK), i32),
        pltpu.VMEM((CHUNK, 2 * D), f32),
        pltpu.VMEM((CHUNK, 2 * D), f32),
        pltpu.VMEM((CHUNK, 2 * D), f32),
        pltpu.VMEM((BPW,), f32),
        pltpu.VMEM((BPW,), f32),
        pltpu.SemaphoreType.DMA,
    ]
    run = pl.kernel(_bprmf_body, out_type=out_type, mesh=mesh,
                    scratch_types=scratch,
                    compiler_params=pltpu.CompilerParams(
                        needs_layout_passes=False,
                        use_tc_tiling_on_sc=True))
    return run(users2, pos2, neg2, ut2, it2)


def kernel(users, pos_items, neg_items, user_table, item_table):
    users2 = users.astype(jnp.int32).reshape(NW * NCH, CHUNK)
    pos2 = pos_items.astype(jnp.int32).reshape(NW * NCH, CHUNK)
    neg2 = neg_items.astype(jnp.int32).reshape(NW * NCH, CHUNK)
    ut2, it2 = _repack(user_table.T, item_table.T)
    ps, ns, uv, pv, nv = _bprmf(users2, pos2, neg2, ut2, it2)
    u_odd = ((users.astype(jnp.int32) >> 6) & 1)[:, None] == 1
    p_odd = ((pos_items.astype(jnp.int32) >> 6) & 1)[:, None] == 1
    n_odd = ((neg_items.astype(jnp.int32) >> 6) & 1)[:, None] == 1
    u_emb = jnp.where(u_odd, uv[:, D:], uv[:, :D])
    pos_emb = jnp.where(p_odd, pv[:, D:], pv[:, :D])
    neg_emb = jnp.where(n_odd, nv[:, D:], nv[:, :D])
    return (ps, ns, u_emb, pos_emb, neg_emb)


# speedup vs baseline: 7.9086x; 1.0584x over previous
import jax
import jax.numpy as jnp
from jax import lax
from jax.experimental import pallas as pl
from jax.experimental.pallas import tpu as pltpu
from jax.experimental.pallas import tpu_sc as plsc

B = 16384
D = 64
NC = 2
NS = 16
NW = NC * NS
BPW = B // NW
CHUNK = 128
NCH = BPW // CHUNK
GPC = CHUNK // 16

N_U = 1000000
N_I = 100000
CB_U = (N_U + 127) // 128
CB_I = (N_I + 127) // 128


def _transpose_stripe(colbuf, rowbuf):
    iota16 = lax.iota(jnp.int32, 16)
    lanes_l0 = [l0 + iota16 for l0 in range(0, 128, 16)]
    v_l0 = [(l0 + iota16) & 63 for l0 in range(0, 128, 16)]
    hb_l0 = [(l0 // 64) * D for l0 in range(0, 128, 16)]

    def r_body(r, _):
        for j in range(2):
            rowoff = (iota16 + (2 * r + j)) & 15
            for d0 in range(0, D, 16):
                rows = rowoff + d0
                vals = [plsc.load_gather(colbuf, [rows, lanes_l0[q]])
                        for q in range(8)]
                for q in range(8):
                    plsc.store_scatter(rowbuf, [v_l0[q], rows + hb_l0[q]],
                                       vals[q])
        return 0

    lax.fori_loop(0, 8, r_body, 0)


def _repack_body(ut_t, it_t, u2_out, i2_out,
                 colA, colB, colC, colD, rowA, rowB, rowC, rowD,
                 semA, semB, semC, semD):
    c = lax.axis_index("c")
    s = lax.axis_index("s")
    wid = s * NC + c
    cols = (colA, colB, colC, colD)
    rows_ = (rowA, rowB, rowC, rowD)
    sems = (semA, semB, semC, semD)

    def make_quad_loop(src, dst, n_full):
        def quad_body(tp, _):
            cbs = []
            cis = []
            for b in range(4):
                cb = jnp.minimum(wid + (4 * tp + b) * NW, n_full - 1)
                st = pl.multiple_of(cb * 128, 128)
                cis.append(pltpu.async_copy(src.at[:, pl.ds(st, 128)],
                                            cols[b], sems[b]))
                cbs.append(cb)
            cos = []
            for b in range(4):
                cis[b].wait()
                _transpose_stripe(cols[b], rows_[b])
                cos.append(pltpu.async_copy(
                    rows_[b], dst.at[pl.ds(cbs[b] * 64, 64)], sems[b]))
            for co in cos:
                co.wait()
            return 0

        return quad_body

    n_full_u = N_U // 128
    n_full_i = N_I // 128
    lax.fori_loop(0, (n_full_u + 4 * NW - 1) // (4 * NW),
                  make_quad_loop(ut_t, u2_out, n_full_u), 0)
    lax.fori_loop(0, (n_full_i + 4 * NW - 1) // (4 * NW),
                  make_quad_loop(it_t, i2_out, n_full_i), 0)

    def tail(src, dst, cb, n_rows, who):
        @pl.when(wid == who)
        def _():
            start = pl.multiple_of(cb * 128, 128)
            pltpu.sync_copy(src.at[:, pl.ds(start, 128)], colA)
            _transpose_stripe(colA, rowA)
            nv = n_rows - cb * 128
            for q in range(4):
                @pl.when(nv >= (q + 1) * 16)
                def _(q=q):
                    pltpu.sync_copy(rowA.at[pl.ds(q * 16, 16)],
                                    dst.at[pl.ds(cb * 64 + q * 16, 16)])

    tail(ut_t, u2_out, n_full_u, N_U, 0)
    tail(it_t, i2_out, n_full_i, N_I, 1)


@jax.jit
def _repack(ut_t, it_t):
    mesh = plsc.VectorSubcoreMesh(core_axis_name="c", subcore_axis_name="s",
                                  num_cores=NC, num_subcores=NS)
    f32 = jnp.float32
    run = pl.kernel(
        _repack_body,
        out_type=(jax.ShapeDtypeStruct((CB_U * 64, 128), f32),
                  jax.ShapeDtypeStruct((CB_I * 64, 128), f32)),
        mesh=mesh,
        scratch_types=[
            pltpu.VMEM((D, 128), f32),
            pltpu.VMEM((D, 128), f32),
            pltpu.VMEM((D, 128), f32),
            pltpu.VMEM((D, 128), f32),
            pltpu.VMEM((D, 128), f32),
            pltpu.VMEM((D, 128), f32),
            pltpu.VMEM((D, 128), f32),
            pltpu.VMEM((D, 128), f32),
            pltpu.SemaphoreType.DMA,
            pltpu.SemaphoreType.DMA,
            pltpu.SemaphoreType.DMA,
            pltpu.SemaphoreType.DMA,
        ],
        compiler_params=pltpu.CompilerParams(
            needs_layout_passes=False,
            use_tc_tiling_on_sc=True,
            disable_bounds_checks=True),
    )
    return run(ut_t, it_t)


def _bprmf_body(users_r, pos_r, neg_r, ut_r, it_r,
                pos_s_out, neg_s_out, u_out, p_out, n_out,
                idx_u, idx_p, idx_n, vr_u, vr_p, vr_n,
                u_rows, p_rows, n_rows, sc_p, sc_n, sem):
    c = lax.axis_index("c")
    s = lax.axis_index("s")
    wid = s * NC + c
    base = wid * BPW

    pltpu.sync_copy(users_r.at[pl.ds(wid * NCH, NCH)], idx_u)
    pltpu.sync_copy(pos_r.at[pl.ds(wid * NCH, NCH)], idx_p)
    pltpu.sync_copy(neg_r.at[pl.ds(wid * NCH, NCH)], idx_n)

    for ch in range(NCH):
        for g in range(GPC):
            sl = pl.ds(g * 16, 16)
            vr_u[ch, sl] = ((idx_u[ch, sl] >> 7) << 6) + (idx_u[ch, sl] & 63)
            vr_p[ch, sl] = ((idx_p[ch, sl] >> 7) << 6) + (idx_p[ch, sl] & 63)
            vr_n[ch, sl] = ((idx_n[ch, sl] >> 7) << 6) + (idx_n[ch, sl] & 63)

    iota16 = lax.iota(jnp.int32, 16)
    zero16 = jnp.zeros((16,), jnp.float32)

    for ch in range(NCH):
        cps = (pltpu.async_copy(ut_r.at[vr_u.at[ch]], u_rows, sem),
               pltpu.async_copy(it_r.at[vr_p.at[ch]], p_rows, sem),
               pltpu.async_copy(it_r.at[vr_n.at[ch]], n_rows, sem))
        for cp in cps:
            cp.wait()

        for g in range(GPC):
            sl = pl.ds(g * 16, 16)
            rows_e = g * 16 + iota16
            hu = ((idx_u[ch, sl] >> 6) & 1) * D
            hp = ((idx_p[ch, sl] >> 6) & 1) * D
            hn = ((idx_n[ch, sl] >> 6) & 1) * D

            def dbody(d, carry, rows_e=rows_e, hu=hu, hp=hp, hn=hn):
                ap, an = carry
                uc = plsc.load_gather(u_rows, [rows_e, hu + d])
                pc = plsc.load_gather(p_rows, [rows_e, hp + d])
                nc = plsc.load_gather(n_rows, [rows_e, hn + d])
                return (ap + uc * pc, an + uc * nc)

            ap, an = lax.fori_loop(0, D, dbody, (zero16, zero16))
            osl = pl.ds(ch * CHUNK + g * 16, 16)
            sc_p[osl] = ap
            sc_n[osl] = an

        out_sl = pl.ds(base + ch * CHUNK, CHUNK)
        pltpu.sync_copy(u_rows, u_out.at[out_sl])
        pltpu.sync_copy(p_rows, p_out.at[out_sl])
        pltpu.sync_copy(n_rows, n_out.at[out_sl])

    out_sl = pl.ds(base, BPW)
    pltpu.sync_copy(sc_p, pos_s_out.at[out_sl])
    pltpu.sync_copy(sc_n, neg_s_out.at[out_sl])


@jax.jit
def _bprmf(users2, pos2, neg2, ut2, it2):
    mesh = plsc.VectorSubcoreMesh(core_axis_name="c", subcore_axis_name="s",
                                  num_cores=NC, num_subcores=NS)
    f32 = jnp.float32
    out_type = (
        jax.ShapeDtypeStruct((B,), f32),
        jax.ShapeDtypeStruct((B,), f32),
        jax.ShapeDtypeStruct((B, 2 * D), f32),
        jax.ShapeDtypeStruct((B, 2 * D), f32),
        jax.ShapeDtypeStruct((B, 2 * D), f32),
    )
    i32 = jnp.int32
    scratch = [
        pltpu.VMEM((NCH, CHUNK), i32),
        pltpu.VMEM((NCH, CHUNK), i32),
        pltpu.VMEM((NCH, CHUNK), i32),
        pltpu.VMEM((NCH, CHUNK), i32),
        pltpu.VMEM((NCH, CHUNK), i32),
        pltpu.VMEM((NCH, CHUNK), i32),
        pltpu.VMEM((CHUNK, 2 * D), f32),
        pltpu.VMEM((CHUNK, 2 * D), f32),
        pltpu.VMEM((CHUNK, 2 * D), f32),
        pltpu.VMEM((BPW,), f32),
        pltpu.VMEM((BPW,), f32),
        pltpu.SemaphoreType.DMA,
    ]
    run = pl.kernel(_bprmf_body, out_type=out_type, mesh=mesh,
                    scratch_types=scratch,
                    compiler_params=pltpu.CompilerParams(
                        needs_layout_passes=False,
                        use_tc_tiling_on_sc=True))
    return run(users2, pos2, neg2, ut2, it2)


def kernel(users, pos_items, neg_items, user_table, item_table):
    users2 = users.astype(jnp.int32).reshape(NW * NCH, CHUNK)
    pos2 = pos_items.astype(jnp.int32).reshape(NW * NCH, CHUNK)
    neg2 = neg_items.astype(jnp.int32).reshape(NW * NCH, CHUNK)
    ut2, it2 = _repack(user_table.T, item_table.T)
    ps, ns, uv, pv, nv = _bprmf(users2, pos2, neg2, ut2, it2)
    u_odd = ((users.astype(jnp.int32) >> 6) & 1)[:, None] == 1
    p_odd = ((pos_items.astype(jnp.int32) >> 6) & 1)[:, None] == 1
    n_odd = ((neg_items.astype(jnp.int32) >> 6) & 1)[:, None] == 1
    u_emb = jnp.where(u_odd, uv[:, D:], uv[:, :D])
    pos_emb = jnp.where(p_odd, pv[:, D:], pv[:, :D])
    neg_emb = jnp.where(n_odd, nv[:, D:], nv[:, :D])
    return (ps, ns, u_emb, pos_emb, neg_emb)
